# Initial kernel scaffold; baseline (speedup 1.0000x reference)
import jax
import jax.numpy as jnp
from jax.experimental import pallas as pl


def kernel(start_ids, table):
    raise NotImplementedError("write your pallas kernel here")


# speedup vs baseline: 4.4969x; 4.4969x over previous
import functools

import jax
import jax.numpy as jnp
from jax import lax
from jax.experimental import pallas as pl
from jax.experimental.pallas import tpu as pltpu
from jax.experimental.pallas import tpu_sc as plsc

VOCAB = 100000
EMBED_DIM = 64
BATCH = 4096
HIST = 50

NUM_CORES = 2
NUM_SUBCORES = 16
NUM_WORKERS = NUM_CORES * NUM_SUBCORES

TOTAL = BATCH * HIST
PER_WORKER = TOTAL // NUM_WORKERS
CHUNK = 640
NUM_CHUNKS = PER_WORKER // CHUNK


_mesh = plsc.VectorSubcoreMesh(core_axis_name="c", subcore_axis_name="s")


@functools.partial(
    pl.kernel,
    out_type=jax.ShapeDtypeStruct((TOTAL, EMBED_DIM), jnp.float32),
    mesh=_mesh,
    scratch_types=[
        pltpu.VMEM((CHUNK,), jnp.int32),
        pltpu.VMEM((CHUNK, EMBED_DIM), jnp.float32),
        pltpu.SemaphoreType.DMA,
    ],
    compiler_params=pltpu.CompilerParams(use_tc_tiling_on_sc=False),
)
def _gather_kernel(ids_hbm, table_hbm, out_hbm, idx_v, rows_v, sem):
    wid = lax.axis_index("s") * NUM_CORES + lax.axis_index("c")
    base = wid * PER_WORKER

    def chunk_body(j, carry):
        off = base + j * CHUNK
        pltpu.sync_copy(ids_hbm.at[pl.ds(off, CHUNK)], idx_v)
        pltpu.async_copy(table_hbm.at[idx_v], rows_v, sem).wait()
        pltpu.sync_copy(rows_v, out_hbm.at[pl.ds(off, CHUNK)])
        return carry

    lax.fori_loop(0, NUM_CHUNKS, chunk_body, 0)


def kernel(start_ids, table):
    ids = start_ids.reshape(-1).astype(jnp.int32)
    out = _gather_kernel(ids, table)
    return out.reshape(BATCH, HIST, EMBED_DIM)


# speedup vs baseline: 4.6596x; 1.0362x over previous
import functools

import jax
import jax.numpy as jnp
from jax import lax
from jax.experimental import pallas as pl
from jax.experimental.pallas import tpu as pltpu
from jax.experimental.pallas import tpu_sc as plsc

VOCAB = 100000
EMBED_DIM = 64
BATCH = 4096
HIST = 50

NUM_CORES = 2
NUM_SUBCORES = 16
NUM_WORKERS = NUM_CORES * NUM_SUBCORES

TOTAL = BATCH * HIST
PER_WORKER = TOTAL // NUM_WORKERS
CHUNK = 800
NUM_CHUNKS = PER_WORKER // CHUNK
NBUF = 2


_mesh = plsc.VectorSubcoreMesh(core_axis_name="c", subcore_axis_name="s")


@functools.partial(
    pl.kernel,
    out_type=jax.ShapeDtypeStruct((TOTAL, EMBED_DIM), jnp.float32),
    mesh=_mesh,
    scratch_types=(
        [pltpu.VMEM((CHUNK,), jnp.int32) for _ in range(NBUF)]
        + [pltpu.VMEM((CHUNK, EMBED_DIM), jnp.float32) for _ in range(NBUF)]
        + [pltpu.SemaphoreType.DMA for _ in range(3 * NBUF)]
    ),
    compiler_params=pltpu.CompilerParams(use_tc_tiling_on_sc=False),
)
def _gather_kernel(ids_hbm, table_hbm, out_hbm,
                   idx0, idx1, rows0, rows1,
                   si0, si1, sg0, sg1, ss0, ss1):
    wid = lax.axis_index("s") * NUM_CORES + lax.axis_index("c")
    base = wid * PER_WORKER

    idx = (idx0, idx1)
    rows = (rows0, rows1)
    si = (si0, si1)
    sg = (sg0, sg1)
    ss = (ss0, ss1)

    def off(c):
        return base + c * CHUNK

    idx_d, g_d, s_d = {}, {}, {}
    for c in range(min(NBUF, NUM_CHUNKS)):
        b = c % NBUF
        idx_d[c] = pltpu.async_copy(
            ids_hbm.at[pl.ds(off(c), CHUNK)], idx[b], si[b])
    idx_d[0].wait()
    g_d[0] = pltpu.async_copy(table_hbm.at[idx[0]], rows[0], sg[0])

    for c in range(NUM_CHUNKS):
        b = c % NBUF
        b2 = (c + 1) % NBUF
        if c + 1 < NUM_CHUNKS:
            idx_d[c + 1].wait()
            if c - 1 >= 0:
                s_d[c - 1].wait()
            g_d[c + 1] = pltpu.async_copy(
                table_hbm.at[idx[b2]], rows[b2], sg[b2])
        g_d[c].wait()
        s_d[c] = pltpu.async_copy(
            rows[b], out_hbm.at[pl.ds(off(c), CHUNK)], ss[b])
        if c + 2 < NUM_CHUNKS:
            idx_d[c + 2] = pltpu.async_copy(
                ids_hbm.at[pl.ds(off(c + 2), CHUNK)], idx[b], si[b])

    if NUM_CHUNKS >= 2:
        s_d[NUM_CHUNKS - 2].wait()
    s_d[NUM_CHUNKS - 1].wait()


def kernel(start_ids, table):
    ids = start_ids.reshape(-1).astype(jnp.int32)
    out = _gather_kernel(ids, table)
    return out.reshape(BATCH, HIST, EMBED_DIM)
